# Initial kernel scaffold; baseline (speedup 1.0000x reference)
import jax
import jax.numpy as jnp
from jax.experimental import pallas as pl


def kernel(x, edge_index, edge_weight, W, b):
    raise NotImplementedError("write your pallas kernel here")


# speedup vs baseline: 4.2594x; 4.2594x over previous
import functools

import jax
import jax.numpy as jnp
from jax import lax
from jax.experimental import pallas as pl
from jax.experimental.pallas import tpu as pltpu
from jax.experimental.pallas import tpu_sc as plsc

NC = 2
NS = 16
L = 16
CH = 128


def _linear(x, Wt, b2):
    M, Din = x.shape
    Dout = Wt.shape[1]
    BM = 1000

    def body(x_ref, wt_ref, b_ref, o_ref):
        o_ref[...] = (
            jnp.dot(x_ref[...], wt_ref[...], preferred_element_type=jnp.float32)
            + b_ref[...]
        )

    return pl.pallas_call(
        body,
        grid=(M // BM,),
        in_specs=[
            pl.BlockSpec((BM, Din), lambda i: (i, 0)),
            pl.BlockSpec((Din, Dout), lambda i: (0, 0)),
            pl.BlockSpec((1, Dout), lambda i: (0, 0)),
        ],
        out_specs=pl.BlockSpec((BM, Dout), lambda i: (i, 0)),
        out_shape=jax.ShapeDtypeStruct((M, Dout), jnp.float32),
    )(x, Wt, b2)


def _combine_relu(p0, p1, n):
    Np, D = p0.shape
    BM = 1000

    def body(a_ref, b_ref, o_ref):
        o_ref[...] = jnp.maximum(a_ref[...] + b_ref[...], 0.0)

    return pl.pallas_call(
        body,
        grid=(n // BM,),
        in_specs=[
            pl.BlockSpec((BM, D), lambda i: (i, 0)),
            pl.BlockSpec((BM, D), lambda i: (i, 0)),
        ],
        out_specs=pl.BlockSpec((BM, D), lambda i: (i, 0)),
        out_shape=jax.ShapeDtypeStruct((n, D), jnp.float32),
    )(p0, p1)


def _spmm_sc(h, src4, dst4, wflat3, n_pad):
    D = h.shape[1]
    K = src4.shape[2]
    RZ = n_pad // (NS * CH)
    mesh = plsc.VectorSubcoreMesh(core_axis_name="c", subcore_axis_name="s")

    @functools.partial(
        pl.kernel,
        mesh=mesh,
        out_type=jax.ShapeDtypeStruct((NC, n_pad, D), jnp.float32),
        scratch_types=[
            pltpu.VMEM((K, CH), jnp.int32),
            pltpu.VMEM((K, CH), jnp.int32),
            pltpu.VMEM((K * CH,), jnp.float32),
            pltpu.VMEM((CH, D), jnp.float32),
            pltpu.VMEM_SHARED((n_pad, D), jnp.float32),
            pltpu.SemaphoreType.DMA,
        ],
    )
    def spmm(src_hbm, dst_hbm, w_hbm, h_hbm, out_hbm,
             src_v, dst_v, w_v, rows_v, acc_sh, sem):
        c = lax.axis_index("c")
        s = lax.axis_index("s")

        pltpu.sync_copy(src_hbm.at[c, s], src_v)
        pltpu.sync_copy(dst_hbm.at[c, s], dst_v)
        pltpu.sync_copy(w_hbm.at[c, s], w_v)

        def zrow(i, _):
            for chk in range(D // L):
                rows_v[i, pl.ds(chk * L, L)] = jnp.zeros((L,), jnp.float32)
            return 0

        lax.fori_loop(0, CH, zrow, 0)
        base = s * (RZ * CH)
        for r in range(RZ):
            pltpu.sync_copy(rows_v, acc_sh.at[pl.ds(base + r * CH, CH)])
        plsc.subcore_barrier()

        def chunk_body(j, _):
            pltpu.async_copy(h_hbm.at[src_v.at[j]], rows_v, sem).wait()

            dnums = lax.GatherDimensionNumbers(
                offset_dims=(), collapsed_slice_dims=(0,), start_index_map=(0,)
            )

            def scale_group(g, _):
                wgrp = w_v[pl.ds(j * CH + g * L, L)]

                def scale_edge(i, _):
                    e = g * L + i
                    bidx = jnp.full((L, 1), i, dtype=jnp.int32)
                    wvec = lax.gather(
                        wgrp, bidx, dnums, (1,),
                        mode=lax.GatherScatterMode.PROMISE_IN_BOUNDS,
                    )
                    for chk in range(D // L):
                        sl = pl.ds(chk * L, L)
                        rows_v[e, sl] = rows_v[e, sl] * wvec
                    return 0

                lax.fori_loop(0, L, scale_edge, 0)
                return 0

            lax.fori_loop(0, CH // L, scale_group, 0)
            pltpu.sync_copy(rows_v, acc_sh.at[dst_v.at[j]], add=True)
            return 0

        lax.fori_loop(0, K, chunk_body, 0)
        plsc.subcore_barrier()

        pltpu.sync_copy(
            acc_sh.at[pl.ds(base, RZ * CH)],
            out_hbm.at[c, pl.ds(base, RZ * CH)],
        )

    return spmm(src4, dst4, wflat3, h)


def kernel(x, edge_index, edge_weight, W, b):
    n, d_in = x.shape
    d_out = W.shape[0]
    e = edge_weight.shape[0]

    h = _linear(x, W.T, b.reshape(1, d_out))

    ew = NC * NS * CH
    k = -(-e // ew)
    e_pad = k * ew
    pad = e_pad - e
    src = jnp.concatenate([edge_index[0], jnp.zeros((pad,), jnp.int32)])
    dst = jnp.concatenate([edge_index[1], jnp.zeros((pad,), jnp.int32)])
    w = jnp.concatenate([edge_weight, jnp.zeros((pad,), jnp.float32)])
    src4 = src.reshape(NC, NS, k, CH)
    dst4 = dst.reshape(NC, NS, k, CH)
    wflat3 = w.reshape(NC, NS, k * CH)

    n_pad = -(-n // (NS * CH)) * (NS * CH)
    partial = _spmm_sc(h, src4, dst4, wflat3, n_pad)

    return _combine_relu(partial[0], partial[1], n)
